# Initial kernel scaffold; baseline (speedup 1.0000x reference)
import functools

import jax
import jax.numpy as jnp
from jax import lax
from jax.experimental import pallas as pl
from jax.experimental.pallas import tpu as pltpu
from jax.experimental.pallas import tpu_sc as plsc

N = 50000
E = 800000
D_IN = 128
D_HID = 64
D_OUT = 128

NC = 2
NS = 16
CG = 32
CHUNK = 80
RPS = N // NS
ZCH = 125

_MESH = dict(core_axis_name="c", subcore_axis_name="s")


def _fill_const(ref, n_rows, n_cols, value):
    @pl.loop(0, n_rows)
    def _(i):
        for c0 in range(0, n_cols, 16):
            ref[i, pl.ds(c0, 16)] = jnp.full((16,), value, jnp.float32)


def _deg_body(dst_hbm, out_hbm, didx, didx_t, ones, zbuf, acc):
    c = lax.axis_index("c")
    s = lax.axis_index("s")
    w = c * NS + s
    epw = E // (NC * NS)
    n_full = epw // CHUNK
    tail = epw - n_full * CHUNK

    _fill_const(ones, CHUNK, 16, 1.0)
    _fill_const(zbuf, ZCH, 16, 0.0)

    @pl.loop(0, RPS // ZCH)
    def _(k):
        pltpu.sync_copy(zbuf, acc.at[pl.ds(s * RPS + k * ZCH, ZCH)])

    plsc.subcore_barrier()

    @pl.loop(0, n_full)
    def _(j):
        base = w * epw + j * CHUNK
        pltpu.sync_copy(dst_hbm.at[pl.ds(base, CHUNK)], didx)
        pltpu.sync_copy(ones, acc.at[didx], add=True)

    base = w * epw + n_full * CHUNK
    pltpu.sync_copy(dst_hbm.at[pl.ds(base, tail)], didx_t)
    pltpu.sync_copy(ones.at[pl.ds(0, tail)], acc.at[didx_t], add=True)

    plsc.subcore_barrier()
    pltpu.sync_copy(acc.at[pl.ds(s * RPS, RPS)],
                    out_hbm.at[pl.ds(c * N + s * RPS, RPS)])


def _deg_partials(dst):
    kern = pl.kernel(
        _deg_body,
        out_type=jax.ShapeDtypeStruct((NC * N, 16), jnp.float32),
        mesh=plsc.VectorSubcoreMesh(**_MESH),
        scratch_types=[
            pltpu.VMEM((CHUNK,), jnp.int32),
            pltpu.VMEM((40,), jnp.int32),
            pltpu.VMEM((CHUNK, 16), jnp.float32),
            pltpu.VMEM((ZCH, 16), jnp.float32),
            pltpu.VMEM_SHARED((N, 16), jnp.float32),
        ],
    )
    return kern(dst)


def _agg_body(gpc, src_hbm, dst_hbm, g_hbm, out_hbm,
              sidx, didx, rows, zbuf, acc):
    c = lax.axis_index("c")
    s = lax.axis_index("s")
    epw = E // NS
    n_ch = epw // CHUNK

    _fill_const(zbuf, ZCH, CG, 0.0)

    for g in range(gpc):
        goff = (c * gpc + g) * N

        @pl.loop(0, RPS // ZCH)
        def _(k):
            pltpu.sync_copy(zbuf, acc.at[pl.ds(s * RPS + k * ZCH, ZCH)])

        plsc.subcore_barrier()

        @pl.loop(0, n_ch)
        def _(j):
            base = s * epw + j * CHUNK
            pltpu.sync_copy(src_hbm.at[pl.ds(base, CHUNK)], sidx)

            @pl.loop(0, CHUNK // 16)
            def _(i):
                sidx[pl.ds(i * 16, 16)] = sidx[pl.ds(i * 16, 16)] + goff

            pltpu.sync_copy(g_hbm.at[sidx], rows)
            pltpu.sync_copy(dst_hbm.at[pl.ds(base, CHUNK)], didx)
            pltpu.sync_copy(rows, acc.at[didx], add=True)

        plsc.subcore_barrier()
        pltpu.sync_copy(acc.at[pl.ds(s * RPS, RPS)],
                        out_hbm.at[pl.ds(goff + s * RPS, RPS)])


def _aggregate(src, dst, g_flat, gpc):
    kern = pl.kernel(
        functools.partial(_agg_body, gpc),
        out_type=jax.ShapeDtypeStruct((gpc * NC * N, CG), jnp.float32),
        mesh=plsc.VectorSubcoreMesh(**_MESH),
        scratch_types=[
            pltpu.VMEM((CHUNK,), jnp.int32),
            pltpu.VMEM((CHUNK,), jnp.int32),
            pltpu.VMEM((CHUNK, CG), jnp.float32),
            pltpu.VMEM((ZCH, CG), jnp.float32),
            pltpu.VMEM_SHARED((N, CG), jnp.float32),
        ],
    )
    return kern(src, dst, g_flat)


_NB = 2000
_HIGH = jax.lax.Precision.HIGHEST


def _mm1_body(x_ref, w_ref, o_ref):
    o_ref[...] = jax.lax.dot(x_ref[...], w_ref[...],
                             precision=_HIGH,
                             preferred_element_type=jnp.float32)


def _mm1(x, W1):
    return pl.pallas_call(
        _mm1_body,
        grid=(N // _NB,),
        in_specs=[pl.BlockSpec((_NB, D_IN), lambda i: (i, 0)),
                  pl.BlockSpec((D_IN, D_HID), lambda i: (0, 0))],
        out_specs=pl.BlockSpec((_NB, D_HID), lambda i: (i, 0)),
        out_shape=jax.ShapeDtypeStruct((N, D_HID), jnp.float32),
    )(x, W1)


def _scale_body(h_ref, degp_ref, g_ref, dinv_ref):
    deg = degp_ref[0, :, 0:1] + degp_ref[1, :, 0:1] + 1.0
    dv = jax.lax.rsqrt(deg)
    dinv_ref[...] = jnp.broadcast_to(dv, (_NB, 16))
    g = h_ref[...] * dv
    g_ref[0] = g[:, :CG]
    g_ref[1] = g[:, CG:]


def _scale(h1, degp):
    return pl.pallas_call(
        _scale_body,
        grid=(N // _NB,),
        in_specs=[pl.BlockSpec((_NB, D_HID), lambda i: (i, 0)),
                  pl.BlockSpec((2, _NB, 16), lambda i: (0, i, 0))],
        out_specs=[pl.BlockSpec((2, _NB, CG), lambda i: (0, i, 0)),
                   pl.BlockSpec((_NB, 16), lambda i: (i, 0))],
        out_shape=[jax.ShapeDtypeStruct((2, N, CG), jnp.float32),
                   jax.ShapeDtypeStruct((N, 16), jnp.float32)],
    )(h1, degp)


def _mid_body(agg_ref, g1_ref, dinv_ref, w2_ref, b1_ref, g2_ref):
    a = jnp.concatenate([agg_ref[0] + g1_ref[0], agg_ref[1] + g1_ref[1]],
                        axis=1)
    dv = dinv_ref[:, 0:1]
    h = jnp.maximum(a * dv + b1_ref[...][None, :], 0.0)
    g2 = jax.lax.dot(h, w2_ref[...], precision=_HIGH,
                     preferred_element_type=jnp.float32) * dv
    for k in range(4):
        g2_ref[k] = g2[:, CG * k:CG * (k + 1)]


def _mid(agg1, g1, dinv, W2, b1):
    return pl.pallas_call(
        _mid_body,
        grid=(N // _NB,),
        in_specs=[pl.BlockSpec((2, _NB, CG), lambda i: (0, i, 0)),
                  pl.BlockSpec((2, _NB, CG), lambda i: (0, i, 0)),
                  pl.BlockSpec((_NB, 16), lambda i: (i, 0)),
                  pl.BlockSpec((D_HID, D_OUT), lambda i: (0, 0)),
                  pl.BlockSpec((D_HID,), lambda i: (0,))],
        out_specs=pl.BlockSpec((4, _NB, CG), lambda i: (0, i, 0)),
        out_shape=jax.ShapeDtypeStruct((4, N, CG), jnp.float32),
    )(agg1, g1, dinv, W2, b1)


def _final_body(agg_ref, g2_ref, dinv_ref, b2_ref, o_ref):
    cat = jnp.concatenate([agg_ref[k] + g2_ref[k] for k in range(4)], axis=1)
    dv = dinv_ref[:, 0:1]
    o_ref[...] = cat * dv + b2_ref[...][None, :]


def _final(agg2, g2, dinv, b2):
    return pl.pallas_call(
        _final_body,
        grid=(N // _NB,),
        in_specs=[pl.BlockSpec((4, _NB, CG), lambda i: (0, i, 0)),
                  pl.BlockSpec((4, _NB, CG), lambda i: (0, i, 0)),
                  pl.BlockSpec((_NB, 16), lambda i: (i, 0)),
                  pl.BlockSpec((D_OUT,), lambda i: (0,))],
        out_specs=pl.BlockSpec((_NB, D_OUT), lambda i: (i, 0)),
        out_shape=jax.ShapeDtypeStruct((N, D_OUT), jnp.float32),
    )(agg2, g2, dinv, b2)


@jax.jit
def _run(x, edge_index, W1, b1, W2, b2):
    src = edge_index[0]
    dst = edge_index[1]

    degp = _deg_partials(dst)
    h1 = _mm1(x, W1)
    g1, dinv = _scale(h1, degp.reshape(2, N, 16))
    agg1 = _aggregate(src, dst, g1.reshape(2 * N, CG), 1)
    g2 = _mid(agg1.reshape(2, N, CG), g1, dinv, W2, b1)
    agg2 = _aggregate(src, dst, g2.reshape(4 * N, CG), 2)
    return _final(agg2.reshape(4, N, CG), g2, dinv, b2)


def kernel(x, edge_index, W1, b1, W2, b2):
    assert x.shape == (N, D_IN) and edge_index.shape == (2, E)
    return _run(x, edge_index, W1, b1, W2, b2)


# speedup vs baseline: 6.1701x; 6.1701x over previous
import functools

import jax
import jax.numpy as jnp
from jax import lax
from jax.experimental import pallas as pl
from jax.experimental.pallas import tpu as pltpu
from jax.experimental.pallas import tpu_sc as plsc

N = 50000
E = 800000
D_IN = 128
D_HID = 64
D_OUT = 128

NC = 2
NS = 16
CG = 32
CHUNK = 80
NP = 50048
RPS = NP // NS
ZCH = 136

_MESH = dict(core_axis_name="c", subcore_axis_name="s")


def _fill_const(ref, n_rows, n_cols, value):
    @pl.loop(0, n_rows)
    def _(i):
        for c0 in range(0, n_cols, 16):
            ref[i, pl.ds(c0, 16)] = jnp.full((16,), value, jnp.float32)


def _deg_body(dst_hbm, out_hbm, didx, ones, zbuf, acc):
    c = lax.axis_index("c")
    s = lax.axis_index("s")

    @pl.when(c == 0)
    def _():
        epw = E // NS
        n_ch = epw // CHUNK

        _fill_const(ones, CHUNK, 16, 1.0)
        _fill_const(zbuf, ZCH, 16, 0.0)

        @pl.loop(0, RPS // ZCH)
        def _(k):
            pltpu.sync_copy(zbuf, acc.at[pl.ds(s * RPS + k * ZCH, ZCH)])

        plsc.subcore_barrier()

        @pl.loop(0, n_ch)
        def _(j):
            base = s * epw + j * CHUNK
            pltpu.sync_copy(dst_hbm.at[pl.ds(base, CHUNK)], didx)
            pltpu.sync_copy(ones, acc.at[didx], add=True)

        plsc.subcore_barrier()
        pltpu.sync_copy(acc.at[pl.ds(s * RPS, RPS)],
                        out_hbm.at[pl.ds(s * RPS, RPS)])


def _deg_partials(dst):
    kern = pl.kernel(
        _deg_body,
        out_type=jax.ShapeDtypeStruct((NP, 16), jnp.float32),
        mesh=plsc.VectorSubcoreMesh(**_MESH),
        scratch_types=[
            pltpu.VMEM((CHUNK,), jnp.int32),
            pltpu.VMEM((CHUNK, 16), jnp.float32),
            pltpu.VMEM((ZCH, 16), jnp.float32),
            pltpu.VMEM_SHARED((NP, 16), jnp.float32),
        ],
        compiler_params=pltpu.CompilerParams(use_tc_tiling_on_sc=False),
    )
    return kern(dst)


def _agg_body(gpc, src_hbm, dst_hbm, g_hbm, out_hbm,
              sidx, didx, rows, zbuf, acc):
    c = lax.axis_index("c")
    s = lax.axis_index("s")
    epw = E // NS
    n_ch = epw // CHUNK

    _fill_const(zbuf, ZCH, CG, 0.0)

    for g in range(gpc):
        grp = c * gpc + g
        goff = grp * N

        @pl.loop(0, RPS // ZCH)
        def _(k):
            pltpu.sync_copy(zbuf, acc.at[pl.ds(s * RPS + k * ZCH, ZCH)])

        plsc.subcore_barrier()

        @pl.loop(0, n_ch)
        def _(j):
            base = s * epw + j * CHUNK
            pltpu.sync_copy(src_hbm.at[pl.ds(base, CHUNK)], sidx)

            @pl.loop(0, CHUNK // 16)
            def _(i):
                sidx[pl.ds(i * 16, 16)] = sidx[pl.ds(i * 16, 16)] + goff

            pltpu.sync_copy(g_hbm.at[sidx], rows)
            pltpu.sync_copy(dst_hbm.at[pl.ds(base, CHUNK)], didx)
            pltpu.sync_copy(rows, acc.at[didx], add=True)

        plsc.subcore_barrier()
        pltpu.sync_copy(acc.at[pl.ds(s * RPS, RPS)],
                        out_hbm.at[pl.ds(grp * NP + s * RPS, RPS)])


def _aggregate(src, dst, g_flat, gpc):
    kern = pl.kernel(
        functools.partial(_agg_body, gpc),
        out_type=jax.ShapeDtypeStruct((gpc * NC * NP, CG), jnp.float32),
        mesh=plsc.VectorSubcoreMesh(**_MESH),
        scratch_types=[
            pltpu.VMEM((CHUNK,), jnp.int32),
            pltpu.VMEM((CHUNK,), jnp.int32),
            pltpu.VMEM((CHUNK, CG), jnp.float32),
            pltpu.VMEM((ZCH, CG), jnp.float32),
            pltpu.VMEM_SHARED((NP, CG), jnp.float32),
        ],
        compiler_params=pltpu.CompilerParams(use_tc_tiling_on_sc=False),
    )
    return kern(src, dst, g_flat)


_NB = 2000
_HIGH = jax.lax.Precision.HIGHEST


def _mm1_body(x_ref, w_ref, o_ref):
    o_ref[...] = jax.lax.dot(x_ref[...], w_ref[...],
                             precision=_HIGH,
                             preferred_element_type=jnp.float32)


def _mm1(x, W1):
    return pl.pallas_call(
        _mm1_body,
        grid=(N // _NB,),
        in_specs=[pl.BlockSpec((_NB, D_IN), lambda i: (i, 0)),
                  pl.BlockSpec((D_IN, D_HID), lambda i: (0, 0))],
        out_specs=pl.BlockSpec((_NB, D_HID), lambda i: (i, 0)),
        out_shape=jax.ShapeDtypeStruct((N, D_HID), jnp.float32),
    )(x, W1)


def _scale_body(h_ref, degp_ref, g_ref, dinv_ref):
    deg = degp_ref[:, 0:1] + 1.0
    dv = jax.lax.rsqrt(deg)
    dinv_ref[...] = jnp.broadcast_to(dv, (_NB, 16))
    g = h_ref[...] * dv
    g_ref[0] = g[:, :CG]
    g_ref[1] = g[:, CG:]


def _scale(h1, degp):
    return pl.pallas_call(
        _scale_body,
        grid=(N // _NB,),
        in_specs=[pl.BlockSpec((_NB, D_HID), lambda i: (i, 0)),
                  pl.BlockSpec((_NB, 16), lambda i: (i, 0))],
        out_specs=[pl.BlockSpec((2, _NB, CG), lambda i: (0, i, 0)),
                   pl.BlockSpec((_NB, 16), lambda i: (i, 0))],
        out_shape=[jax.ShapeDtypeStruct((2, N, CG), jnp.float32),
                   jax.ShapeDtypeStruct((N, 16), jnp.float32)],
    )(h1, degp)


def _mid_body(agg_ref, g1_ref, dinv_ref, w2_ref, b1_ref, g2_ref):
    a = jnp.concatenate([agg_ref[0] + g1_ref[0], agg_ref[1] + g1_ref[1]],
                        axis=1)
    dv = dinv_ref[:, 0:1]
    h = jnp.maximum(a * dv + b1_ref[...][None, :], 0.0)
    g2 = jax.lax.dot(h, w2_ref[...], precision=_HIGH,
                     preferred_element_type=jnp.float32) * dv
    for k in range(4):
        g2_ref[k] = g2[:, CG * k:CG * (k + 1)]


def _mid(agg1, g1, dinv, W2, b1):
    return pl.pallas_call(
        _mid_body,
        grid=(N // _NB,),
        in_specs=[pl.BlockSpec((2, _NB, CG), lambda i: (0, i, 0)),
                  pl.BlockSpec((2, _NB, CG), lambda i: (0, i, 0)),
                  pl.BlockSpec((_NB, 16), lambda i: (i, 0)),
                  pl.BlockSpec((D_HID, D_OUT), lambda i: (0, 0)),
                  pl.BlockSpec((D_HID,), lambda i: (0,))],
        out_specs=pl.BlockSpec((4, _NB, CG), lambda i: (0, i, 0)),
        out_shape=jax.ShapeDtypeStruct((4, N, CG), jnp.float32),
    )(agg1, g1, dinv, W2, b1)


def _final_body(agg_ref, g2_ref, dinv_ref, b2_ref, o_ref):
    cat = jnp.concatenate([agg_ref[k] + g2_ref[k] for k in range(4)], axis=1)
    dv = dinv_ref[:, 0:1]
    o_ref[...] = cat * dv + b2_ref[...][None, :]


def _final(agg2, g2, dinv, b2):
    return pl.pallas_call(
        _final_body,
        grid=(N // _NB,),
        in_specs=[pl.BlockSpec((4, _NB, CG), lambda i: (0, i, 0)),
                  pl.BlockSpec((4, _NB, CG), lambda i: (0, i, 0)),
                  pl.BlockSpec((_NB, 16), lambda i: (i, 0)),
                  pl.BlockSpec((D_OUT,), lambda i: (0,))],
        out_specs=pl.BlockSpec((_NB, D_OUT), lambda i: (i, 0)),
        out_shape=jax.ShapeDtypeStruct((N, D_OUT), jnp.float32),
    )(agg2, g2, dinv, b2)


@jax.jit
def _run(x, edge_index, W1, b1, W2, b2):
    src = edge_index[0]
    dst = edge_index[1]

    degp = _deg_partials(dst)
    h1 = _mm1(x, W1)
    g1, dinv = _scale(h1, degp)
    agg1 = _aggregate(src, dst, g1.reshape(2 * N, CG), 1)
    g2 = _mid(agg1.reshape(2, NP, CG), g1, dinv, W2, b1)
    agg2 = _aggregate(src, dst, g2.reshape(4 * N, CG), 2)
    return _final(agg2.reshape(4, NP, CG), g2, dinv, b2)


def kernel(x, edge_index, W1, b1, W2, b2):
    assert x.shape == (N, D_IN) and edge_index.shape == (2, E)
    return _run(x, edge_index, W1, b1, W2, b2)


# speedup vs baseline: 16.4510x; 2.6662x over previous
import functools

import jax
import jax.numpy as jnp
from jax import lax
from jax.experimental import pallas as pl
from jax.experimental.pallas import tpu as pltpu
from jax.experimental.pallas import tpu_sc as plsc

N = 50000
E = 800000
D_IN = 128
D_HID = 64
D_OUT = 128

NC = 2
NS = 16
CG = 32
CHUNK = 80
NP = 50048
RPS = NP // NS
ZCH = 136

_MESH = dict(core_axis_name="c", subcore_axis_name="s")


def _fill_const(ref, n_rows, n_cols, value):
    @pl.loop(0, n_rows)
    def _(i):
        for c0 in range(0, n_cols, 16):
            ref[i, pl.ds(c0, 16)] = jnp.full((16,), value, jnp.float32)


BLK = 25
CPS = (E // CHUNK) // NS


def _deg_body(dst2_hbm, out_hbm, didx, ones, zbuf, acc, ssem):
    c = lax.axis_index("c")
    s = lax.axis_index("s")

    @pl.when(c == 0)
    def _():
        _fill_const(ones, CHUNK, 16, 1.0)
        _fill_const(zbuf, ZCH, 16, 0.0)

        @pl.loop(0, RPS // ZCH)
        def _(k):
            pltpu.sync_copy(zbuf, acc.at[pl.ds(s * RPS + k * ZCH, ZCH)])

        plsc.subcore_barrier()

        @pl.loop(0, CPS // BLK)
        def _(r):
            row0 = s * CPS + r * BLK
            pltpu.sync_copy(dst2_hbm.at[pl.ds(row0, BLK)], didx)
            scats = [pltpu.async_copy(ones, acc.at[didx.at[j]], ssem,
                                      add=True)
                     for j in range(BLK)]
            for sc in scats:
                sc.wait()

        plsc.subcore_barrier()
        pltpu.sync_copy(acc.at[pl.ds(s * RPS, RPS)],
                        out_hbm.at[pl.ds(s * RPS, RPS)])


def _deg_partials(dst2):
    kern = pl.kernel(
        _deg_body,
        out_type=jax.ShapeDtypeStruct((NP, 16), jnp.float32),
        mesh=plsc.VectorSubcoreMesh(**_MESH),
        scratch_types=[
            pltpu.VMEM((BLK, CHUNK), jnp.int32),
            pltpu.VMEM((CHUNK, 16), jnp.float32),
            pltpu.VMEM((ZCH, 16), jnp.float32),
            pltpu.VMEM_SHARED((NP, 16), jnp.float32),
            pltpu.SemaphoreType.DMA,
        ],
        compiler_params=pltpu.CompilerParams(use_tc_tiling_on_sc=False),
    )
    return kern(dst2)


def _agg_body(gpc, src2_hbm, dst2_hbm, g_hbm, out_hbm,
              sidx, didx, rows0, rows1, zbuf, acc, gsem, ssem):
    c = lax.axis_index("c")
    s = lax.axis_index("s")
    rows = [rows0, rows1]

    _fill_const(zbuf, ZCH, CG, 0.0)

    for g in range(gpc):
        grp = c * gpc + g
        goff = grp * N

        @pl.loop(0, RPS // ZCH)
        def _(k):
            pltpu.sync_copy(zbuf, acc.at[pl.ds(s * RPS + k * ZCH, ZCH)])

        plsc.subcore_barrier()

        @pl.loop(0, CPS // BLK)
        def _(r):
            row0 = s * CPS + r * BLK
            pltpu.sync_copy(src2_hbm.at[pl.ds(row0, BLK)], sidx)
            pltpu.sync_copy(dst2_hbm.at[pl.ds(row0, BLK)], didx)

            @pl.loop(0, BLK)
            def _(j):
                for i in range(CHUNK // 16):
                    sidx[j, pl.ds(i * 16, 16)] = (
                        sidx[j, pl.ds(i * 16, 16)] + goff)

            gathers = [pltpu.async_copy(g_hbm.at[sidx.at[0]], rows[0],
                                        gsem.at[0])]
            scat = [None, None]
            for j in range(BLK):
                b = j & 1
                if j + 1 < BLK:
                    bn = (j + 1) & 1
                    if scat[bn] is not None:
                        scat[bn].wait()
                        scat[bn] = None
                    gathers.append(
                        pltpu.async_copy(g_hbm.at[sidx.at[j + 1]], rows[bn],
                                         gsem.at[bn]))
                gathers[j].wait()
                scat[b] = pltpu.async_copy(rows[b], acc.at[didx.at[j]],
                                           ssem.at[b], add=True)
            for t in range(2):
                if scat[t] is not None:
                    scat[t].wait()

        plsc.subcore_barrier()
        pltpu.sync_copy(acc.at[pl.ds(s * RPS, RPS)],
                        out_hbm.at[pl.ds(grp * NP + s * RPS, RPS)])


def _aggregate(src2, dst2, g_flat, gpc):
    kern = pl.kernel(
        functools.partial(_agg_body, gpc),
        out_type=jax.ShapeDtypeStruct((gpc * NC * NP, CG), jnp.float32),
        mesh=plsc.VectorSubcoreMesh(**_MESH),
        scratch_types=[
            pltpu.VMEM((BLK, CHUNK), jnp.int32),
            pltpu.VMEM((BLK, CHUNK), jnp.int32),
            pltpu.VMEM((CHUNK, CG), jnp.float32),
            pltpu.VMEM((CHUNK, CG), jnp.float32),
            pltpu.VMEM((ZCH, CG), jnp.float32),
            pltpu.VMEM_SHARED((NP, CG), jnp.float32),
            pltpu.SemaphoreType.DMA((2,)),
            pltpu.SemaphoreType.DMA((2,)),
        ],
        compiler_params=pltpu.CompilerParams(use_tc_tiling_on_sc=False),
    )
    return kern(src2, dst2, g_flat)


_NB = 2000
_HIGH = jax.lax.Precision.HIGHEST


def _mm1_body(x_ref, w_ref, o_ref):
    o_ref[...] = jax.lax.dot(x_ref[...], w_ref[...],
                             precision=_HIGH,
                             preferred_element_type=jnp.float32)


def _mm1(x, W1):
    return pl.pallas_call(
        _mm1_body,
        grid=(N // _NB,),
        in_specs=[pl.BlockSpec((_NB, D_IN), lambda i: (i, 0)),
                  pl.BlockSpec((D_IN, D_HID), lambda i: (0, 0))],
        out_specs=pl.BlockSpec((_NB, D_HID), lambda i: (i, 0)),
        out_shape=jax.ShapeDtypeStruct((N, D_HID), jnp.float32),
    )(x, W1)


def _scale_body(h_ref, degp_ref, g_ref, dinv_ref):
    deg = degp_ref[:, 0:1] + 1.0
    dv = jax.lax.rsqrt(deg)
    dinv_ref[...] = jnp.broadcast_to(dv, (_NB, 16))
    g = h_ref[...] * dv
    g_ref[0] = g[:, :CG]
    g_ref[1] = g[:, CG:]


def _scale(h1, degp):
    return pl.pallas_call(
        _scale_body,
        grid=(N // _NB,),
        in_specs=[pl.BlockSpec((_NB, D_HID), lambda i: (i, 0)),
                  pl.BlockSpec((_NB, 16), lambda i: (i, 0))],
        out_specs=[pl.BlockSpec((2, _NB, CG), lambda i: (0, i, 0)),
                   pl.BlockSpec((_NB, 16), lambda i: (i, 0))],
        out_shape=[jax.ShapeDtypeStruct((2, N, CG), jnp.float32),
                   jax.ShapeDtypeStruct((N, 16), jnp.float32)],
    )(h1, degp)


def _mid_body(agg_ref, g1_ref, dinv_ref, w2_ref, b1_ref, g2_ref):
    a = jnp.concatenate([agg_ref[0] + g1_ref[0], agg_ref[1] + g1_ref[1]],
                        axis=1)
    dv = dinv_ref[:, 0:1]
    h = jnp.maximum(a * dv + b1_ref[...][None, :], 0.0)
    g2 = jax.lax.dot(h, w2_ref[...], precision=_HIGH,
                     preferred_element_type=jnp.float32) * dv
    for k in range(4):
        g2_ref[k] = g2[:, CG * k:CG * (k + 1)]


def _mid(agg1, g1, dinv, W2, b1):
    return pl.pallas_call(
        _mid_body,
        grid=(N // _NB,),
        in_specs=[pl.BlockSpec((2, _NB, CG), lambda i: (0, i, 0)),
                  pl.BlockSpec((2, _NB, CG), lambda i: (0, i, 0)),
                  pl.BlockSpec((_NB, 16), lambda i: (i, 0)),
                  pl.BlockSpec((D_HID, D_OUT), lambda i: (0, 0)),
                  pl.BlockSpec((D_HID,), lambda i: (0,))],
        out_specs=pl.BlockSpec((4, _NB, CG), lambda i: (0, i, 0)),
        out_shape=jax.ShapeDtypeStruct((4, N, CG), jnp.float32),
    )(agg1, g1, dinv, W2, b1)


def _final_body(agg_ref, g2_ref, dinv_ref, b2_ref, o_ref):
    cat = jnp.concatenate([agg_ref[k] + g2_ref[k] for k in range(4)], axis=1)
    dv = dinv_ref[:, 0:1]
    o_ref[...] = cat * dv + b2_ref[...][None, :]


def _final(agg2, g2, dinv, b2):
    return pl.pallas_call(
        _final_body,
        grid=(N // _NB,),
        in_specs=[pl.BlockSpec((4, _NB, CG), lambda i: (0, i, 0)),
                  pl.BlockSpec((4, _NB, CG), lambda i: (0, i, 0)),
                  pl.BlockSpec((_NB, 16), lambda i: (i, 0)),
                  pl.BlockSpec((D_OUT,), lambda i: (0,))],
        out_specs=pl.BlockSpec((_NB, D_OUT), lambda i: (i, 0)),
        out_shape=jax.ShapeDtypeStruct((N, D_OUT), jnp.float32),
    )(agg2, g2, dinv, b2)


@jax.jit
def _run(x, edge_index, W1, b1, W2, b2):
    src2 = edge_index[0].reshape(E // CHUNK, CHUNK)
    dst2 = edge_index[1].reshape(E // CHUNK, CHUNK)

    degp = _deg_partials(dst2)
    h1 = _mm1(x, W1)
    g1, dinv = _scale(h1, degp)
    agg1 = _aggregate(src2, dst2, g1.reshape(2 * N, CG), 1)
    g2 = _mid(agg1.reshape(2, NP, CG), g1, dinv, W2, b1)
    agg2 = _aggregate(src2, dst2, g2.reshape(4 * N, CG), 2)
    return _final(agg2.reshape(4, NP, CG), g2, dinv, b2)


def kernel(x, edge_index, W1, b1, W2, b2):
    assert x.shape == (N, D_IN) and edge_index.shape == (2, E)
    return _run(x, edge_index, W1, b1, W2, b2)


# speedup vs baseline: 21.6251x; 1.3145x over previous
import functools

import jax
import jax.numpy as jnp
from jax import lax
from jax.experimental import pallas as pl
from jax.experimental.pallas import tpu as pltpu
from jax.experimental.pallas import tpu_sc as plsc

N = 50000
E = 800000
D_IN = 128
D_HID = 64
D_OUT = 128

NC = 2
NS = 16
CG = 32
CHUNK = 80
NP = 50048
RPS = NP // NS
ZCH = 136

_MESH = dict(core_axis_name="c", subcore_axis_name="s")


def _fill_const(ref, n_rows, n_cols, value):
    @pl.loop(0, n_rows)
    def _(i):
        for c0 in range(0, n_cols, 16):
            ref[i, pl.ds(c0, 16)] = jnp.full((16,), value, jnp.float32)


BLK = 25
CPS = (E // CHUNK) // NS


def _deg_body(dst2_hbm, out_hbm, didx, ones, zbuf, acc, ssem):
    c = lax.axis_index("c")
    s = lax.axis_index("s")

    @pl.when(c == 0)
    def _():
        _fill_const(ones, CHUNK, 16, 1.0)
        _fill_const(zbuf, ZCH, 16, 0.0)

        @pl.loop(0, RPS // ZCH)
        def _(k):
            pltpu.sync_copy(zbuf, acc.at[pl.ds(s * RPS + k * ZCH, ZCH)])

        plsc.subcore_barrier()

        @pl.loop(0, CPS // BLK)
        def _(r):
            row0 = s * CPS + r * BLK
            pltpu.sync_copy(dst2_hbm.at[pl.ds(row0, BLK)], didx)
            scats = [pltpu.async_copy(ones, acc.at[didx.at[j]], ssem,
                                      add=True)
                     for j in range(BLK)]
            for sc in scats:
                sc.wait()

        plsc.subcore_barrier()
        pltpu.sync_copy(acc.at[pl.ds(s * RPS, RPS)],
                        out_hbm.at[pl.ds(s * RPS, RPS)])


def _deg_partials(dst2):
    kern = pl.kernel(
        _deg_body,
        out_type=jax.ShapeDtypeStruct((NP, 16), jnp.float32),
        mesh=plsc.VectorSubcoreMesh(**_MESH),
        scratch_types=[
            pltpu.VMEM((BLK, CHUNK), jnp.int32),
            pltpu.VMEM((CHUNK, 16), jnp.float32),
            pltpu.VMEM((ZCH, 16), jnp.float32),
            pltpu.VMEM_SHARED((NP, 16), jnp.float32),
            pltpu.SemaphoreType.DMA,
        ],
        compiler_params=pltpu.CompilerParams(use_tc_tiling_on_sc=False),
    )
    return kern(dst2)


ABLK = 25
NBUF = 4
LOOKAHEAD = 3


def _agg_body(gpc, src2_hbm, dst2_hbm, g_hbm, out_hbm,
              sidx, didx, rows0, rows1, rows2, rows3, zbuf, acc, gsem, ssem):
    c = lax.axis_index("c")
    s = lax.axis_index("s")
    rows = [rows0, rows1, rows2, rows3]

    _fill_const(zbuf, ZCH, CG, 0.0)

    for g in range(gpc):
        grp = c * gpc + g
        goff = grp * N

        @pl.loop(0, RPS // ZCH)
        def _(k):
            pltpu.sync_copy(zbuf, acc.at[pl.ds(s * RPS + k * ZCH, ZCH)])

        plsc.subcore_barrier()

        @pl.loop(0, CPS // ABLK)
        def _(r):
            row0 = s * CPS + r * ABLK
            pltpu.sync_copy(src2_hbm.at[pl.ds(row0, ABLK)], sidx)
            pltpu.sync_copy(dst2_hbm.at[pl.ds(row0, ABLK)], didx)

            @pl.loop(0, ABLK)
            def _(j):
                for i in range(CHUNK // 16):
                    sidx[j, pl.ds(i * 16, 16)] = (
                        sidx[j, pl.ds(i * 16, 16)] + goff)

            gathers = [None] * NBUF
            scat = [None] * NBUF
            for j in range(LOOKAHEAD):
                gathers[j % NBUF] = pltpu.async_copy(
                    g_hbm.at[sidx.at[j]], rows[j % NBUF], gsem.at[j % NBUF])
            for j in range(ABLK):
                b = j % NBUF
                jn = j + LOOKAHEAD
                if jn < ABLK:
                    bn = jn % NBUF
                    if scat[bn] is not None:
                        scat[bn].wait()
                        scat[bn] = None
                    gathers[bn] = pltpu.async_copy(
                        g_hbm.at[sidx.at[jn]], rows[bn], gsem.at[bn])
                gathers[b].wait()
                scat[b] = pltpu.async_copy(rows[b], acc.at[didx.at[j]],
                                           ssem.at[b], add=True)
            for t in range(NBUF):
                if scat[t] is not None:
                    scat[t].wait()

        plsc.subcore_barrier()
        pltpu.sync_copy(acc.at[pl.ds(s * RPS, RPS)],
                        out_hbm.at[pl.ds(grp * NP + s * RPS, RPS)])


def _aggregate(src2, dst2, g_flat, gpc):
    kern = pl.kernel(
        functools.partial(_agg_body, gpc),
        out_type=jax.ShapeDtypeStruct((gpc * NC * NP, CG), jnp.float32),
        mesh=plsc.VectorSubcoreMesh(**_MESH),
        scratch_types=[
            pltpu.VMEM((ABLK, CHUNK), jnp.int32),
            pltpu.VMEM((ABLK, CHUNK), jnp.int32),
            pltpu.VMEM((CHUNK, CG), jnp.float32),
            pltpu.VMEM((CHUNK, CG), jnp.float32),
            pltpu.VMEM((CHUNK, CG), jnp.float32),
            pltpu.VMEM((CHUNK, CG), jnp.float32),
            pltpu.VMEM((ZCH, CG), jnp.float32),
            pltpu.VMEM_SHARED((NP, CG), jnp.float32),
            pltpu.SemaphoreType.DMA((NBUF,)),
            pltpu.SemaphoreType.DMA((NBUF,)),
        ],
        compiler_params=pltpu.CompilerParams(use_tc_tiling_on_sc=False),
    )
    return kern(src2, dst2, g_flat)


_NB = 2000
_HIGH = jax.lax.Precision.HIGHEST


def _mm1_body(x_ref, w_ref, o_ref):
    o_ref[...] = jax.lax.dot(x_ref[...], w_ref[...],
                             precision=_HIGH,
                             preferred_element_type=jnp.float32)


def _mm1(x, W1):
    return pl.pallas_call(
        _mm1_body,
        grid=(N // _NB,),
        in_specs=[pl.BlockSpec((_NB, D_IN), lambda i: (i, 0)),
                  pl.BlockSpec((D_IN, D_HID), lambda i: (0, 0))],
        out_specs=pl.BlockSpec((_NB, D_HID), lambda i: (i, 0)),
        out_shape=jax.ShapeDtypeStruct((N, D_HID), jnp.float32),
    )(x, W1)


def _scale_body(h_ref, degp_ref, g_ref, dinv_ref):
    deg = degp_ref[:, 0:1] + 1.0
    dv = jax.lax.rsqrt(deg)
    dinv_ref[...] = jnp.broadcast_to(dv, (_NB, 16))
    g = h_ref[...] * dv
    g_ref[0] = g[:, :CG]
    g_ref[1] = g[:, CG:]


def _scale(h1, degp):
    return pl.pallas_call(
        _scale_body,
        grid=(N // _NB,),
        in_specs=[pl.BlockSpec((_NB, D_HID), lambda i: (i, 0)),
                  pl.BlockSpec((_NB, 16), lambda i: (i, 0))],
        out_specs=[pl.BlockSpec((2, _NB, CG), lambda i: (0, i, 0)),
                   pl.BlockSpec((_NB, 16), lambda i: (i, 0))],
        out_shape=[jax.ShapeDtypeStruct((2, N, CG), jnp.float32),
                   jax.ShapeDtypeStruct((N, 16), jnp.float32)],
    )(h1, degp)


def _mid_body(agg_ref, g1_ref, dinv_ref, w2_ref, b1_ref, g2_ref):
    a = jnp.concatenate([agg_ref[0] + g1_ref[0], agg_ref[1] + g1_ref[1]],
                        axis=1)
    dv = dinv_ref[:, 0:1]
    h = jnp.maximum(a * dv + b1_ref[...][None, :], 0.0)
    g2 = jax.lax.dot(h, w2_ref[...], precision=_HIGH,
                     preferred_element_type=jnp.float32) * dv
    for k in range(4):
        g2_ref[k] = g2[:, CG * k:CG * (k + 1)]


def _mid(agg1, g1, dinv, W2, b1):
    return pl.pallas_call(
        _mid_body,
        grid=(N // _NB,),
        in_specs=[pl.BlockSpec((2, _NB, CG), lambda i: (0, i, 0)),
                  pl.BlockSpec((2, _NB, CG), lambda i: (0, i, 0)),
                  pl.BlockSpec((_NB, 16), lambda i: (i, 0)),
                  pl.BlockSpec((D_HID, D_OUT), lambda i: (0, 0)),
                  pl.BlockSpec((D_HID,), lambda i: (0,))],
        out_specs=pl.BlockSpec((4, _NB, CG), lambda i: (0, i, 0)),
        out_shape=jax.ShapeDtypeStruct((4, N, CG), jnp.float32),
    )(agg1, g1, dinv, W2, b1)


def _final_body(agg_ref, g2_ref, dinv_ref, b2_ref, o_ref):
    cat = jnp.concatenate([agg_ref[k] + g2_ref[k] for k in range(4)], axis=1)
    dv = dinv_ref[:, 0:1]
    o_ref[...] = cat * dv + b2_ref[...][None, :]


def _final(agg2, g2, dinv, b2):
    return pl.pallas_call(
        _final_body,
        grid=(N // _NB,),
        in_specs=[pl.BlockSpec((4, _NB, CG), lambda i: (0, i, 0)),
                  pl.BlockSpec((4, _NB, CG), lambda i: (0, i, 0)),
                  pl.BlockSpec((_NB, 16), lambda i: (i, 0)),
                  pl.BlockSpec((D_OUT,), lambda i: (0,))],
        out_specs=pl.BlockSpec((_NB, D_OUT), lambda i: (i, 0)),
        out_shape=jax.ShapeDtypeStruct((N, D_OUT), jnp.float32),
    )(agg2, g2, dinv, b2)


@jax.jit
def _run(x, edge_index, W1, b1, W2, b2):
    src2 = edge_index[0].reshape(E // CHUNK, CHUNK)
    dst2 = edge_index[1].reshape(E // CHUNK, CHUNK)

    degp = _deg_partials(dst2)
    h1 = _mm1(x, W1)
    g1, dinv = _scale(h1, degp)
    agg1 = _aggregate(src2, dst2, g1.reshape(2 * N, CG), 1)
    g2 = _mid(agg1.reshape(2, NP, CG), g1, dinv, W2, b1)
    agg2 = _aggregate(src2, dst2, g2.reshape(4 * N, CG), 2)
    return _final(agg2.reshape(4, NP, CG), g2, dinv, b2)


def kernel(x, edge_index, W1, b1, W2, b2):
    assert x.shape == (N, D_IN) and edge_index.shape == (2, E)
    return _run(x, edge_index, W1, b1, W2, b2)
